# baseline (speedup 1.0000x reference)
import functools

import jax
import jax.numpy as jnp
from jax import lax
from jax.experimental import pallas as pl
from jax.experimental.pallas import tpu as pltpu
from jax.experimental.pallas import tpu_sc as plsc

_N = 10000
_H = 128
_E = 320000
_NW = 32
_K = 128
_C = 80
_CB = 8
_NB = _C // _CB
_EW = _E // _NW
_PW = _C * _K - _EW
_NSUB = 16
_NPAD = 10240
_SEG = _NPAD // _NSUB


def _ln_relu(x, g, b):
    m = jnp.mean(x, axis=-1, keepdims=True)
    xc = x - m
    v = jnp.mean(xc * xc, axis=-1, keepdims=True)
    y = xc * lax.rsqrt(v + 1e-5) * g + b
    return jnp.maximum(y, 0.0)


def _enc_body(P, x_ref, w1, b1, g1, be1, w2, b2, g2, be2, w3, b3, g3, be3,
              out_ref):
    p = pl.program_id(0)
    h = jnp.dot(x_ref[...], w1[...], preferred_element_type=jnp.float32) + b1[...]
    h = _ln_relu(h, g1[...], be1[...])
    h = jnp.dot(h, w2[...], preferred_element_type=jnp.float32) + b2[...]
    h = _ln_relu(h, g2[...], be2[...])

    @pl.when(p == 0)
    def _():
        out_ref[...] = h

    @pl.when(p != 0)
    def _():
        out_ref[...] = jnp.maximum(out_ref[...], h)

    @pl.when(p == P - 1)
    def _():
        y = jnp.dot(out_ref[...], w3[...], preferred_element_type=jnp.float32) + b3[...]
        out_ref[...] = _ln_relu(y, g3[...], be3[...])


def _encoder(x2, p, P, F):
    vec = lambda v: v.reshape(1, _H)
    full = lambda r, c: pl.BlockSpec((r, c), lambda i: (0, 0))
    args = (x2,
            p["w1"], vec(p["b1"]), vec(p["g1"]), vec(p["be1"]),
            p["w2"], vec(p["b2"]), vec(p["g2"]), vec(p["be2"]),
            p["w3"], vec(p["b3"]), vec(p["g3"]), vec(p["be3"]))
    in_specs = [pl.BlockSpec((_N, F), lambda i: (i, 0)),
                full(F, _H), full(1, _H), full(1, _H), full(1, _H),
                full(_H, _H), full(1, _H), full(1, _H), full(1, _H),
                full(_H, _H), full(1, _H), full(1, _H), full(1, _H)]
    return pl.pallas_call(
        functools.partial(_enc_body, P),
        grid=(P,),
        in_specs=in_specs,
        out_specs=pl.BlockSpec((_N, _H), lambda i: (0, 0)),
        out_shape=jax.ShapeDtypeStruct((_N, _H), jnp.float32),
    )(*args)


def _sc_mesh():
    return plsc.VectorSubcoreMesh(core_axis_name="c", subcore_axis_name="s")


def _seg_sums(table, src3, dst3, zrow, tok):
    @functools.partial(
        pl.kernel,
        out_type=jax.ShapeDtypeStruct((2, _NPAD, _H), jnp.float32),
        mesh=_sc_mesh(),
        scratch_types=[
            pltpu.VMEM((_CB, _K), jnp.int32),
            pltpu.VMEM((_CB, _K), jnp.int32),
            pltpu.VMEM((_K, _H), jnp.float32),
            pltpu.VMEM_SHARED((_NPAD, _H), jnp.float32),
            pltpu.SemaphoreType.DMA,
        ],
    )
    def k(table_hbm, src_hbm, dst_hbm, zrow_hbm, tok_hbm, sums_hbm,
          src_v, dst_v, rows_v, acc_sh, sem):
        del tok_hbm
        cid = lax.axis_index("c")
        sid = lax.axis_index("s")
        wid = sid * 2 + cid
        pltpu.sync_copy(zrow_hbm, rows_v)

        def zinit(j, carry):
            pltpu.sync_copy(rows_v, acc_sh.at[pl.ds(sid * _SEG + j * _K, _K)])
            return carry

        lax.fori_loop(0, _SEG // _K, zinit, 0)
        plsc.subcore_barrier()

        def blk(b, carry):
            pltpu.sync_copy(src_hbm.at[wid, pl.ds(b * _CB, _CB)], src_v)
            pltpu.sync_copy(dst_hbm.at[wid, pl.ds(b * _CB, _CB)], dst_v)

            def body(c, carry2):
                pltpu.async_copy(table_hbm.at[src_v.at[c]], rows_v, sem).wait()
                pltpu.sync_copy(rows_v, acc_sh.at[dst_v.at[c]], add=True)
                return carry2

            lax.fori_loop(0, _CB, body, 0)
            return carry

        lax.fori_loop(0, _NB, blk, 0)
        plsc.subcore_barrier()

        def zout(j, carry):
            o = sid * _SEG + j * _K
            pltpu.sync_copy(acc_sh.at[pl.ds(o, _K)], rows_v)
            pltpu.sync_copy(rows_v, sums_hbm.at[cid, pl.ds(o, _K)])
            return carry

        lax.fori_loop(0, _SEG // _K, zout, 0)

    return k(table, src3, dst3, zrow, tok)


def _seg_counts(dst3, zrow, ones, tok):
    @functools.partial(
        pl.kernel,
        out_type=jax.ShapeDtypeStruct((2, _NPAD, _H), jnp.float32),
        mesh=_sc_mesh(),
        scratch_types=[
            pltpu.VMEM((_CB, _K), jnp.int32),
            pltpu.VMEM((_K, _H), jnp.float32),
            pltpu.VMEM_SHARED((_NPAD, _H), jnp.float32),
        ],
    )
    def k(dst_hbm, zrow_hbm, ones_hbm, tok_hbm, cnts_hbm,
          dst_v, rows_v, acc_sh):
        del tok_hbm
        cid = lax.axis_index("c")
        sid = lax.axis_index("s")
        wid = sid * 2 + cid
        pltpu.sync_copy(zrow_hbm, rows_v)

        def zinit(j, carry):
            pltpu.sync_copy(rows_v, acc_sh.at[pl.ds(sid * _SEG + j * _K, _K)])
            return carry

        lax.fori_loop(0, _SEG // _K, zinit, 0)
        pltpu.sync_copy(ones_hbm, rows_v)
        plsc.subcore_barrier()

        def blk(b, carry):
            pltpu.sync_copy(dst_hbm.at[wid, pl.ds(b * _CB, _CB)], dst_v)

            def body(c, carry2):
                pltpu.sync_copy(rows_v, acc_sh.at[dst_v.at[c]], add=True)
                return carry2

            lax.fori_loop(0, _CB, body, 0)
            return carry

        lax.fori_loop(0, _NB, blk, 0)
        plsc.subcore_barrier()

        def zout(j, carry):
            o = sid * _SEG + j * _K
            pltpu.sync_copy(acc_sh.at[pl.ds(o, _K)], rows_v)
            pltpu.sync_copy(rows_v, cnts_hbm.at[cid, pl.ds(o, _K)])
            return carry

        lax.fori_loop(0, _SEG // _K, zout, 0)

    return k(dst3, zrow, ones, tok)


def _gnn_mlp_body(node_ref, s_ref, c_ref, w1a, w1b, b1, g1, be1,
                  w2, b2, g2, be2, out_ref):
    s = s_ref[0] + s_ref[1]
    cnt = c_ref[0][:, :1] + c_ref[1][:, :1]
    aggr = s / jnp.maximum(cnt, 1.0)
    node = node_ref[...]
    h = (jnp.dot(node, w1a[...], preferred_element_type=jnp.float32)
         + jnp.dot(aggr, w1b[...], preferred_element_type=jnp.float32)
         + b1[...])
    h = _ln_relu(h, g1[...], be1[...])
    h = jnp.dot(h, w2[...], preferred_element_type=jnp.float32) + b2[...]
    h = _ln_relu(h, g2[...], be2[...])
    out_ref[...] = node + h


def _gnn_mlp(node, sums, cnts, p, block=2000):
    vec = lambda v: v.reshape(1, _H)
    full = lambda r, c: pl.BlockSpec((r, c), lambda i: (0, 0))
    nb = _N // block
    args = (node, sums, cnts,
            p["w1"][:_H], p["w1"][_H:], vec(p["b1"]), vec(p["g1"]), vec(p["be1"]),
            p["w2"], vec(p["b2"]), vec(p["g2"]), vec(p["be2"]))
    in_specs = [pl.BlockSpec((block, _H), lambda i: (i, 0)),
                pl.BlockSpec((2, block, _H), lambda i: (0, i, 0)),
                pl.BlockSpec((2, block, _H), lambda i: (0, i, 0)),
                full(_H, _H), full(_H, _H), full(1, _H), full(1, _H), full(1, _H),
                full(_H, _H), full(1, _H), full(1, _H), full(1, _H)]
    return pl.pallas_call(
        _gnn_mlp_body,
        grid=(nb,),
        in_specs=in_specs,
        out_specs=pl.BlockSpec((block, _H), lambda i: (i, 0)),
        out_shape=jax.ShapeDtypeStruct((_N, _H), jnp.float32),
    )(*args)


def kernel(lane_points, agent_history, edge_lane_lane, edge_agent_agent,
           edge_lane_agent, params):
    lane_pts = lane_points - lane_points[:, -1:, :2]
    agent_ref = agent_history[:, -1:, :2]
    agent_pts = jnp.concatenate(
        [agent_history[:, :, :2] - agent_ref, agent_history[:, :, 2:]], axis=-1)
    lane_x2 = lane_pts.transpose(1, 0, 2).reshape(20 * _N, 2)
    agent_x2 = agent_pts.transpose(1, 0, 2).reshape(50 * _N, 7)

    lane_feat = _encoder(lane_x2, params["lane"], 20, 2)
    agent_feat = _encoder(agent_x2, params["agent"], 50, 7)

    zrow = jnp.zeros((_K, _H), jnp.float32)
    ones = jnp.ones((_K, _H), jnp.float32)

    def pad_edges(edge):
        s = edge[0].reshape(_NW, _EW)
        d = edge[1].reshape(_NW, _EW)
        ps = jnp.broadcast_to((jnp.arange(_PW, dtype=jnp.int32) * 41) % _N,
                              (_NW, _PW))
        pd = jnp.broadcast_to(_N + (jnp.arange(_PW, dtype=jnp.int32) % (_NPAD - _N)),
                              (_NW, _PW))
        src3 = jnp.concatenate([s, ps], axis=1).reshape(_NW, _C, _K)
        dst3 = jnp.concatenate([d, pd], axis=1).reshape(_NW, _C, _K)
        return src3, dst3

    def gnn(node, src_feat, edge, p, tok):
        src3, dst3 = pad_edges(edge)
        sums = _seg_sums(src_feat, src3, dst3, zrow, tok)
        cnts = _seg_counts(dst3, zrow, ones, sums[0, :8])
        return _gnn_mlp(node, sums, cnts, p), cnts[0, :8]

    tok0 = jnp.zeros((8, _H), jnp.float32)
    lane_feat, tok1 = gnn(lane_feat, lane_feat, edge_lane_lane, params["ll"], tok0)
    agent_feat, tok2 = gnn(agent_feat, agent_feat, edge_agent_agent, params["aa"], tok1)
    agent_feat, _ = gnn(agent_feat, lane_feat, edge_lane_agent, params["la"], tok2)
    return (lane_feat, agent_feat)


# speedup vs baseline: 1.0936x; 1.0936x over previous
import functools

import jax
import jax.numpy as jnp
from jax import lax
from jax.experimental import pallas as pl
from jax.experimental.pallas import tpu as pltpu
from jax.experimental.pallas import tpu_sc as plsc

_N = 10000
_H = 128
_E = 320000
_NW = 32
_K = 128
_C = 80
_CB = 8
_NB = _C // _CB
_EW = _E // _NW
_PW = _C * _K - _EW
_NSUB = 16
_NPAD = 10240
_SEG = _NPAD // _NSUB


def _ln_relu(x, g, b):
    m = jnp.mean(x, axis=-1, keepdims=True)
    xc = x - m
    v = jnp.mean(xc * xc, axis=-1, keepdims=True)
    y = xc * lax.rsqrt(v + 1e-5) * g + b
    return jnp.maximum(y, 0.0)


def _enc_body(P, x_ref, w1, b1, g1, be1, w2, b2, g2, be2, w3, b3, g3, be3,
              out_ref):
    p = pl.program_id(0)
    h = jnp.dot(x_ref[...], w1[...], preferred_element_type=jnp.float32) + b1[...]
    h = _ln_relu(h, g1[...], be1[...])
    h = jnp.dot(h, w2[...], preferred_element_type=jnp.float32) + b2[...]
    h = _ln_relu(h, g2[...], be2[...])

    @pl.when(p == 0)
    def _():
        out_ref[...] = h

    @pl.when(p != 0)
    def _():
        out_ref[...] = jnp.maximum(out_ref[...], h)

    @pl.when(p == P - 1)
    def _():
        y = jnp.dot(out_ref[...], w3[...], preferred_element_type=jnp.float32) + b3[...]
        out_ref[...] = _ln_relu(y, g3[...], be3[...])


def _encoder(x2, p, P, F):
    vec = lambda v: v.reshape(1, _H)
    full = lambda r, c: pl.BlockSpec((r, c), lambda i: (0, 0))
    args = (x2,
            p["w1"], vec(p["b1"]), vec(p["g1"]), vec(p["be1"]),
            p["w2"], vec(p["b2"]), vec(p["g2"]), vec(p["be2"]),
            p["w3"], vec(p["b3"]), vec(p["g3"]), vec(p["be3"]))
    in_specs = [pl.BlockSpec((_N, F), lambda i: (i, 0)),
                full(F, _H), full(1, _H), full(1, _H), full(1, _H),
                full(_H, _H), full(1, _H), full(1, _H), full(1, _H),
                full(_H, _H), full(1, _H), full(1, _H), full(1, _H)]
    return pl.pallas_call(
        functools.partial(_enc_body, P),
        grid=(P,),
        in_specs=in_specs,
        out_specs=pl.BlockSpec((_N, _H), lambda i: (0, 0)),
        out_shape=jax.ShapeDtypeStruct((_N, _H), jnp.float32),
    )(*args)


def _sc_mesh():
    return plsc.VectorSubcoreMesh(core_axis_name="c", subcore_axis_name="s")


def _seg_sums(table, src3, dst3, zrow, tok):
    @functools.partial(
        pl.kernel,
        out_type=jax.ShapeDtypeStruct((2, _NPAD, _H), jnp.float32),
        mesh=_sc_mesh(),
        scratch_types=[
            pltpu.VMEM((_CB, _K), jnp.int32),
            pltpu.VMEM((_CB, _K), jnp.int32),
            pltpu.VMEM((_K, _H), jnp.float32),
            pltpu.VMEM((_K, _H), jnp.float32),
            pltpu.VMEM_SHARED((_NPAD, _H), jnp.float32),
            pltpu.SemaphoreType.DMA,
            pltpu.SemaphoreType.DMA,
        ],
    )
    def k(table_hbm, src_hbm, dst_hbm, zrow_hbm, tok_hbm, sums_hbm,
          src_v, dst_v, rows_v, rows_b, acc_sh, sem, sem_b):
        del tok_hbm
        cid = lax.axis_index("c")
        sid = lax.axis_index("s")
        wid = sid * 2 + cid
        pltpu.sync_copy(zrow_hbm, rows_v)

        def zinit(j, carry):
            pltpu.sync_copy(rows_v, acc_sh.at[pl.ds(sid * _SEG + j * _K, _K)])
            return carry

        lax.fori_loop(0, _SEG // _K, zinit, 0)
        plsc.subcore_barrier()

        def blk(b, carry):
            pltpu.sync_copy(src_hbm.at[wid, pl.ds(b * _CB, _CB)], src_v)
            pltpu.sync_copy(dst_hbm.at[wid, pl.ds(b * _CB, _CB)], dst_v)
            pltpu.async_copy(table_hbm.at[src_v.at[0]], rows_v, sem)

            def body(i, carry2):
                c0 = 2 * i
                pltpu.make_async_copy(table_hbm.at[src_v.at[c0]], rows_v, sem).wait()
                pltpu.async_copy(table_hbm.at[src_v.at[c0 + 1]], rows_b, sem_b)
                pltpu.sync_copy(rows_v, acc_sh.at[dst_v.at[c0]], add=True)
                pltpu.make_async_copy(table_hbm.at[src_v.at[c0 + 1]], rows_b, sem_b).wait()

                @pl.when(c0 + 2 < _CB)
                def _():
                    pltpu.async_copy(table_hbm.at[src_v.at[c0 + 2]], rows_v, sem)

                pltpu.sync_copy(rows_b, acc_sh.at[dst_v.at[c0 + 1]], add=True)
                return carry2

            lax.fori_loop(0, _CB // 2, body, 0)
            return carry

        lax.fori_loop(0, _NB, blk, 0)
        plsc.subcore_barrier()

        def zout(j, carry):
            o = sid * _SEG + j * _K
            pltpu.sync_copy(acc_sh.at[pl.ds(o, _K)], rows_v)
            pltpu.sync_copy(rows_v, sums_hbm.at[cid, pl.ds(o, _K)])
            return carry

        lax.fori_loop(0, _SEG // _K, zout, 0)

    return k(table, src3, dst3, zrow, tok)


def _seg_counts(dst3, zrow, ones, tok):
    @functools.partial(
        pl.kernel,
        out_type=jax.ShapeDtypeStruct((2, _NPAD, _H), jnp.float32),
        mesh=_sc_mesh(),
        scratch_types=[
            pltpu.VMEM((_CB, _K), jnp.int32),
            pltpu.VMEM((_K, _H), jnp.float32),
            pltpu.VMEM_SHARED((_NPAD, _H), jnp.float32),
        ],
    )
    def k(dst_hbm, zrow_hbm, ones_hbm, tok_hbm, cnts_hbm,
          dst_v, rows_v, acc_sh):
        del tok_hbm
        cid = lax.axis_index("c")
        sid = lax.axis_index("s")
        wid = sid * 2 + cid
        pltpu.sync_copy(zrow_hbm, rows_v)

        def zinit(j, carry):
            pltpu.sync_copy(rows_v, acc_sh.at[pl.ds(sid * _SEG + j * _K, _K)])
            return carry

        lax.fori_loop(0, _SEG // _K, zinit, 0)
        pltpu.sync_copy(ones_hbm, rows_v)
        plsc.subcore_barrier()

        def blk(b, carry):
            pltpu.sync_copy(dst_hbm.at[wid, pl.ds(b * _CB, _CB)], dst_v)

            def body(c, carry2):
                pltpu.sync_copy(rows_v, acc_sh.at[dst_v.at[c]], add=True)
                return carry2

            lax.fori_loop(0, _CB, body, 0)
            return carry

        lax.fori_loop(0, _NB, blk, 0)
        plsc.subcore_barrier()

        def zout(j, carry):
            o = sid * _SEG + j * _K
            pltpu.sync_copy(acc_sh.at[pl.ds(o, _K)], rows_v)
            pltpu.sync_copy(rows_v, cnts_hbm.at[cid, pl.ds(o, _K)])
            return carry

        lax.fori_loop(0, _SEG // _K, zout, 0)

    return k(dst3, zrow, ones, tok)


def _gnn_mlp_body(node_ref, s_ref, c_ref, w1a, w1b, b1, g1, be1,
                  w2, b2, g2, be2, out_ref):
    s = s_ref[0] + s_ref[1]
    cnt = c_ref[0][:, :1] + c_ref[1][:, :1]
    aggr = s / jnp.maximum(cnt, 1.0)
    node = node_ref[...]
    h = (jnp.dot(node, w1a[...], preferred_element_type=jnp.float32)
         + jnp.dot(aggr, w1b[...], preferred_element_type=jnp.float32)
         + b1[...])
    h = _ln_relu(h, g1[...], be1[...])
    h = jnp.dot(h, w2[...], preferred_element_type=jnp.float32) + b2[...]
    h = _ln_relu(h, g2[...], be2[...])
    out_ref[...] = node + h


def _gnn_mlp(node, sums, cnts, p, block=2000):
    vec = lambda v: v.reshape(1, _H)
    full = lambda r, c: pl.BlockSpec((r, c), lambda i: (0, 0))
    nb = _N // block
    args = (node, sums, cnts,
            p["w1"][:_H], p["w1"][_H:], vec(p["b1"]), vec(p["g1"]), vec(p["be1"]),
            p["w2"], vec(p["b2"]), vec(p["g2"]), vec(p["be2"]))
    in_specs = [pl.BlockSpec((block, _H), lambda i: (i, 0)),
                pl.BlockSpec((2, block, _H), lambda i: (0, i, 0)),
                pl.BlockSpec((2, block, _H), lambda i: (0, i, 0)),
                full(_H, _H), full(_H, _H), full(1, _H), full(1, _H), full(1, _H),
                full(_H, _H), full(1, _H), full(1, _H), full(1, _H)]
    return pl.pallas_call(
        _gnn_mlp_body,
        grid=(nb,),
        in_specs=in_specs,
        out_specs=pl.BlockSpec((block, _H), lambda i: (i, 0)),
        out_shape=jax.ShapeDtypeStruct((_N, _H), jnp.float32),
    )(*args)


def kernel(lane_points, agent_history, edge_lane_lane, edge_agent_agent,
           edge_lane_agent, params):
    lane_pts = lane_points - lane_points[:, -1:, :2]
    agent_ref = agent_history[:, -1:, :2]
    agent_pts = jnp.concatenate(
        [agent_history[:, :, :2] - agent_ref, agent_history[:, :, 2:]], axis=-1)
    lane_x2 = lane_pts.transpose(1, 0, 2).reshape(20 * _N, 2)
    agent_x2 = agent_pts.transpose(1, 0, 2).reshape(50 * _N, 7)

    lane_feat = _encoder(lane_x2, params["lane"], 20, 2)
    agent_feat = _encoder(agent_x2, params["agent"], 50, 7)

    zrow = jnp.zeros((_K, _H), jnp.float32)
    ones = jnp.ones((_K, _H), jnp.float32)

    def pad_edges(edge):
        s = edge[0].reshape(_NW, _EW)
        d = edge[1].reshape(_NW, _EW)
        ps = jnp.broadcast_to((jnp.arange(_PW, dtype=jnp.int32) * 41) % _N,
                              (_NW, _PW))
        pd = jnp.broadcast_to(_N + (jnp.arange(_PW, dtype=jnp.int32) % (_NPAD - _N)),
                              (_NW, _PW))
        src3 = jnp.concatenate([s, ps], axis=1).reshape(_NW, _C, _K)
        dst3 = jnp.concatenate([d, pd], axis=1).reshape(_NW, _C, _K)
        return src3, dst3

    def gnn(node, src_feat, edge, p, tok):
        src3, dst3 = pad_edges(edge)
        sums = _seg_sums(src_feat, src3, dst3, zrow, tok)
        cnts = _seg_counts(dst3, zrow, ones, sums[0, :8])
        return _gnn_mlp(node, sums, cnts, p), cnts[0, :8]

    tok0 = jnp.zeros((8, _H), jnp.float32)
    lane_feat, tok1 = gnn(lane_feat, lane_feat, edge_lane_lane, params["ll"], tok0)
    agent_feat, tok2 = gnn(agent_feat, agent_feat, edge_agent_agent, params["aa"], tok1)
    agent_feat, _ = gnn(agent_feat, lane_feat, edge_lane_agent, params["la"], tok2)
    return (lane_feat, agent_feat)
